# Initial kernel scaffold; baseline (speedup 1.0000x reference)
import functools

import jax
import jax.numpy as jnp
from jax import lax
from jax.experimental import pallas as pl
from jax.experimental.pallas import tpu as pltpu
from jax.experimental.pallas import tpu_sc as plsc

N_NODES = 10000
NFEAT = 256
NHID = 256
E = 160000
NC = 2
NS = 16
L = 16
HALF = NHID // 2
CHUNK = 128
N_CHUNKS = E // CHUNK
ROWS_PER_SUB = N_NODES // NS

_sc_mesh = plsc.VectorSubcoreMesh(core_axis_name="c", subcore_axis_name="s")


def _deg_body(dst_hbm, zeros_hbm, ones_hbm, out_hbm, idx_v, ones_v, deg_sh):
    cid = lax.axis_index("c")
    sid = lax.axis_index("s")
    pltpu.sync_copy(zeros_hbm.at[pl.ds(sid * ROWS_PER_SUB, ROWS_PER_SUB)],
                    deg_sh.at[pl.ds(sid * ROWS_PER_SUB, ROWS_PER_SUB)])
    pltpu.sync_copy(ones_hbm, ones_v)
    plsc.subcore_barrier()

    half_chunks = N_CHUNKS // NC

    @pl.loop(sid, half_chunks, step=NS)
    def _(chunk):
        base = (cid * half_chunks + chunk) * CHUNK
        pltpu.sync_copy(dst_hbm.at[pl.ds(base, CHUNK)], idx_v)
        pltpu.sync_copy(ones_v, deg_sh.at[idx_v], add=True)

    plsc.subcore_barrier()
    pltpu.sync_copy(deg_sh.at[pl.ds(sid * ROWS_PER_SUB, ROWS_PER_SUB)],
                    out_hbm.at[cid].at[pl.ds(sid * ROWS_PER_SUB, ROWS_PER_SUB)])


def _degree_parts(dst):
    zeros = jnp.zeros((N_NODES, L), jnp.float32)
    ones = jnp.ones((CHUNK, L), jnp.float32)
    k = pl.kernel(
        _deg_body,
        out_type=jax.ShapeDtypeStruct((NC, N_NODES, L), jnp.float32),
        mesh=_sc_mesh,
        scratch_types=[
            pltpu.VMEM((CHUNK,), jnp.int32),
            pltpu.VMEM((CHUNK, L), jnp.float32),
            pltpu.VMEM_SHARED((N_NODES, L), jnp.float32),
        ],
    )
    return k(dst, zeros, ones)


_MM_R = 2000


def _mm_body(x_ref, dinv_ref, w_ref, o_ref):
    xs = x_ref[...] * dinv_ref[...]
    o_ref[0] = lax.dot_general(
        xs, w_ref[...], (((1,), (0,)), ((), ())),
        precision=lax.Precision.HIGHEST, preferred_element_type=jnp.float32)


def _matmul_halves(x, dinv2, W):
    return pl.pallas_call(
        _mm_body,
        grid=(N_NODES // _MM_R, NC),
        in_specs=[
            pl.BlockSpec((_MM_R, NFEAT), lambda i, j: (i, 0)),
            pl.BlockSpec((_MM_R, 1), lambda i, j: (i, 0)),
            pl.BlockSpec((NFEAT, HALF), lambda i, j: (0, j)),
        ],
        out_specs=pl.BlockSpec((1, _MM_R, HALF), lambda i, j: (j, i, 0)),
        out_shape=jax.ShapeDtypeStruct((NC, N_NODES, HALF), jnp.float32),
    )(x, dinv2, W)


def _scatter_body(h2_hbm, src_hbm, dst_hbm, out_hbm, sidx_v, didx_v, rows_v,
                  acc_sh):
    cid = lax.axis_index("c")
    sid = lax.axis_index("s")
    pltpu.sync_copy(h2_hbm.at[cid].at[pl.ds(sid * ROWS_PER_SUB, ROWS_PER_SUB)],
                    acc_sh.at[pl.ds(sid * ROWS_PER_SUB, ROWS_PER_SUB)])
    plsc.subcore_barrier()

    @pl.loop(sid, N_CHUNKS, step=NS)
    def _(chunk):
        base = chunk * CHUNK
        pltpu.sync_copy(src_hbm.at[pl.ds(base, CHUNK)], sidx_v)
        pltpu.sync_copy(dst_hbm.at[pl.ds(base, CHUNK)], didx_v)
        pltpu.sync_copy(h2_hbm.at[cid].at[sidx_v], rows_v)
        pltpu.sync_copy(rows_v, acc_sh.at[didx_v], add=True)

    plsc.subcore_barrier()
    pltpu.sync_copy(acc_sh.at[pl.ds(sid * ROWS_PER_SUB, ROWS_PER_SUB)],
                    out_hbm.at[cid].at[pl.ds(sid * ROWS_PER_SUB, ROWS_PER_SUB)])


def _gather_scatter(h2, src, dst):
    k = pl.kernel(
        _scatter_body,
        out_type=jax.ShapeDtypeStruct((NC, N_NODES, HALF), jnp.float32),
        mesh=_sc_mesh,
        scratch_types=[
            pltpu.VMEM((CHUNK,), jnp.int32),
            pltpu.VMEM((CHUNK,), jnp.int32),
            pltpu.VMEM((CHUNK, HALF), jnp.float32),
            pltpu.VMEM_SHARED((N_NODES, HALF), jnp.float32),
        ],
    )
    return k(h2, src, dst)


_EP_R = 2000


def _ep_body(acc_ref, dinv_ref, b_ref, o_ref):
    a = jnp.concatenate([acc_ref[0], acc_ref[1]], axis=1)
    o_ref[...] = jnp.maximum(a * dinv_ref[...] + b_ref[...], 0.0)


def _epilogue(acc, dinv2, b2):
    return pl.pallas_call(
        _ep_body,
        grid=(N_NODES // _EP_R,),
        in_specs=[
            pl.BlockSpec((NC, _EP_R, HALF), lambda i: (0, i, 0)),
            pl.BlockSpec((_EP_R, 1), lambda i: (i, 0)),
            pl.BlockSpec((1, NHID), lambda i: (0, 0)),
        ],
        out_specs=pl.BlockSpec((_EP_R, NHID), lambda i: (i, 0)),
        out_shape=jax.ShapeDtypeStruct((N_NODES, NHID), jnp.float32),
    )(acc, dinv2, b2)


def kernel(x, edge_index, W, b):
    src = edge_index[0].astype(jnp.int32)
    dst = edge_index[1].astype(jnp.int32)

    deg_parts = _degree_parts(dst)
    deg = deg_parts[:, :, 0].sum(axis=0) + 1.0
    dinv2 = (deg ** -0.5)[:, None]

    h2 = _matmul_halves(x, dinv2, W)
    acc = _gather_scatter(h2, src, dst)
    return _epilogue(acc, dinv2, b.reshape(1, NHID))


# speedup vs baseline: 11.7761x; 11.7761x over previous
import functools

import jax
import jax.numpy as jnp
from jax import lax
from jax.experimental import pallas as pl
from jax.experimental.pallas import tpu as pltpu
from jax.experimental.pallas import tpu_sc as plsc

N_NODES = 10000
NFEAT = 256
NHID = 256
E = 160000
NC = 2
NS = 16
L = 16
HALF = NHID // 2
CHUNK = 128
N_CHUNKS = E // CHUNK
ROWS_PER_SUB = 624
TAIL_START = NS * ROWS_PER_SUB
TAIL_ROWS = N_NODES - TAIL_START


def _copy_rows(sid, src_at, dst_at):
    pltpu.sync_copy(src_at(sid * ROWS_PER_SUB, ROWS_PER_SUB),
                    dst_at(sid * ROWS_PER_SUB, ROWS_PER_SUB))

    @pl.when(sid == NS - 1)
    def _():
        pltpu.sync_copy(src_at(TAIL_START, TAIL_ROWS),
                        dst_at(TAIL_START, TAIL_ROWS))

_sc_mesh = plsc.VectorSubcoreMesh(core_axis_name="c", subcore_axis_name="s")


def _deg_body(dst_hbm, zeros_hbm, ones_hbm, out_hbm, idx_v, ones_v, deg_sh):
    cid = lax.axis_index("c")
    sid = lax.axis_index("s")
    _copy_rows(sid, lambda o, n: zeros_hbm.at[pl.ds(o, n)],
               lambda o, n: deg_sh.at[pl.ds(o, n)])
    pltpu.sync_copy(ones_hbm, ones_v)
    plsc.subcore_barrier()

    half_chunks = N_CHUNKS // NC

    @pl.loop(sid, half_chunks, step=NS)
    def _(chunk):
        base = (cid * half_chunks + chunk) * CHUNK
        pltpu.sync_copy(dst_hbm.at[pl.ds(base, CHUNK)], idx_v)
        pltpu.sync_copy(ones_v, deg_sh.at[idx_v], add=True)

    plsc.subcore_barrier()
    _copy_rows(sid, lambda o, n: deg_sh.at[pl.ds(o, n)],
               lambda o, n: out_hbm.at[cid].at[pl.ds(o, n)])


def _degree_parts(dst):
    zeros = jnp.zeros((N_NODES, L), jnp.float32)
    ones = jnp.ones((CHUNK, L), jnp.float32)
    k = pl.kernel(
        _deg_body,
        out_type=jax.ShapeDtypeStruct((NC, N_NODES, L), jnp.float32),
        mesh=_sc_mesh,
        scratch_types=[
            pltpu.VMEM((CHUNK,), jnp.int32),
            pltpu.VMEM((CHUNK, L), jnp.float32),
            pltpu.VMEM_SHARED((N_NODES, L), jnp.float32),
        ],
    )
    return k(dst, zeros, ones)


_MM_R = 2000


def _mm_body(x_ref, dinv_ref, w_ref, o_ref):
    xs = x_ref[...] * dinv_ref[...]
    o_ref[0] = lax.dot_general(
        xs, w_ref[...], (((1,), (0,)), ((), ())),
        precision=lax.Precision.HIGHEST, preferred_element_type=jnp.float32)


def _matmul_halves(x, dinv2, W):
    return pl.pallas_call(
        _mm_body,
        grid=(N_NODES // _MM_R, NC),
        in_specs=[
            pl.BlockSpec((_MM_R, NFEAT), lambda i, j: (i, 0)),
            pl.BlockSpec((_MM_R, 1), lambda i, j: (i, 0)),
            pl.BlockSpec((NFEAT, HALF), lambda i, j: (0, j)),
        ],
        out_specs=pl.BlockSpec((1, _MM_R, HALF), lambda i, j: (j, i, 0)),
        out_shape=jax.ShapeDtypeStruct((NC, N_NODES, HALF), jnp.float32),
    )(x, dinv2, W)


def _scatter_body(h2_hbm, src_hbm, dst_hbm, out_hbm, sidx_v, didx_v, rows_v,
                  acc_sh):
    cid = lax.axis_index("c")
    sid = lax.axis_index("s")
    _copy_rows(sid, lambda o, n: h2_hbm.at[cid].at[pl.ds(o, n)],
               lambda o, n: acc_sh.at[pl.ds(o, n)])
    plsc.subcore_barrier()

    @pl.loop(sid, N_CHUNKS, step=NS)
    def _(chunk):
        base = chunk * CHUNK
        pltpu.sync_copy(src_hbm.at[pl.ds(base, CHUNK)], sidx_v)
        pltpu.sync_copy(dst_hbm.at[pl.ds(base, CHUNK)], didx_v)
        pltpu.sync_copy(h2_hbm.at[cid].at[sidx_v], rows_v)
        pltpu.sync_copy(rows_v, acc_sh.at[didx_v], add=True)

    plsc.subcore_barrier()
    _copy_rows(sid, lambda o, n: acc_sh.at[pl.ds(o, n)],
               lambda o, n: out_hbm.at[cid].at[pl.ds(o, n)])


def _gather_scatter(h2, src, dst):
    k = pl.kernel(
        _scatter_body,
        out_type=jax.ShapeDtypeStruct((NC, N_NODES, HALF), jnp.float32),
        mesh=_sc_mesh,
        scratch_types=[
            pltpu.VMEM((CHUNK,), jnp.int32),
            pltpu.VMEM((CHUNK,), jnp.int32),
            pltpu.VMEM((CHUNK, HALF), jnp.float32),
            pltpu.VMEM_SHARED((N_NODES, HALF), jnp.float32),
        ],
    )
    return k(h2, src, dst)


_EP_R = 2000


def _ep_body(acc_ref, dinv_ref, b_ref, o_ref):
    a = jnp.concatenate([acc_ref[0], acc_ref[1]], axis=1)
    o_ref[...] = jnp.maximum(a * dinv_ref[...] + b_ref[...], 0.0)


def _epilogue(acc, dinv2, b2):
    return pl.pallas_call(
        _ep_body,
        grid=(N_NODES // _EP_R,),
        in_specs=[
            pl.BlockSpec((NC, _EP_R, HALF), lambda i: (0, i, 0)),
            pl.BlockSpec((_EP_R, 1), lambda i: (i, 0)),
            pl.BlockSpec((1, NHID), lambda i: (0, 0)),
        ],
        out_specs=pl.BlockSpec((_EP_R, NHID), lambda i: (i, 0)),
        out_shape=jax.ShapeDtypeStruct((N_NODES, NHID), jnp.float32),
    )(acc, dinv2, b2)


def kernel(x, edge_index, W, b):
    src = edge_index[0].astype(jnp.int32)
    dst = edge_index[1].astype(jnp.int32)

    deg_parts = _degree_parts(dst)
    deg = deg_parts[:, :, 0].sum(axis=0) + 1.0
    dinv2 = (deg ** -0.5)[:, None]

    h2 = _matmul_halves(x, dinv2, W)
    acc = _gather_scatter(h2, src, dst)
    return _epilogue(acc, dinv2, b.reshape(1, NHID))


# speedup vs baseline: 18.4648x; 1.5680x over previous
import jax
import jax.numpy as jnp
from jax import lax
from jax.experimental import pallas as pl
from jax.experimental.pallas import tpu as pltpu
from jax.experimental.pallas import tpu_sc as plsc

N_NODES = 10000
NFEAT = 256
NHID = 256
E = 160000
NC = 2
NS = 16
L = 16
HALF = NHID // 2
CHUNK = 128
N_CHUNKS = E // CHUNK
ROWS_PER_SUB = 624
TAIL_START = NS * ROWS_PER_SUB
TAIL_ROWS = N_NODES - TAIL_START

_sc_mesh = plsc.VectorSubcoreMesh(core_axis_name="c", subcore_axis_name="s")


def _copy_rows(sid, src_at, dst_at):
    pltpu.sync_copy(src_at(sid * ROWS_PER_SUB, ROWS_PER_SUB),
                    dst_at(sid * ROWS_PER_SUB, ROWS_PER_SUB))

    @pl.when(sid == NS - 1)
    def _():
        pltpu.sync_copy(src_at(TAIL_START, TAIL_ROWS),
                        dst_at(TAIL_START, TAIL_ROWS))


def _deg_body(dst_hbm, zeros_hbm, ones_hbm, out_hbm, idx_v, ones_v, deg_sh):
    cid = lax.axis_index("c")
    sid = lax.axis_index("s")
    _copy_rows(sid, lambda o, n: zeros_hbm.at[pl.ds(o, n)],
               lambda o, n: deg_sh.at[pl.ds(o, n)])
    pltpu.sync_copy(ones_hbm, ones_v)
    plsc.subcore_barrier()

    hc = N_CHUNKS // NC

    @pl.loop(sid, hc, step=NS)
    def _(chunk):
        base = (cid * hc + chunk) * CHUNK
        pltpu.sync_copy(dst_hbm.at[pl.ds(base, CHUNK)], idx_v)
        pltpu.sync_copy(ones_v, deg_sh.at[idx_v], add=True)

    plsc.subcore_barrier()
    _copy_rows(sid, lambda o, n: deg_sh.at[pl.ds(o, n)],
               lambda o, n: out_hbm.at[cid].at[pl.ds(o, n)])


def _degree_parts(dst):
    zeros = jnp.zeros((N_NODES, L), jnp.float32)
    ones = jnp.ones((CHUNK, L), jnp.float32)
    k = pl.kernel(
        _deg_body,
        out_type=jax.ShapeDtypeStruct((NC, N_NODES, L), jnp.float32),
        mesh=_sc_mesh,
        scratch_types=[
            pltpu.VMEM((CHUNK,), jnp.int32),
            pltpu.VMEM((CHUNK, L), jnp.float32),
            pltpu.VMEM_SHARED((N_NODES, L), jnp.float32),
        ],
    )
    return k(dst, zeros, ones)


_MM_R = 2000


def _mm_body(x_ref, dinv_ref, w_ref, o_ref):
    xs = x_ref[...] * dinv_ref[...]
    o_ref[0] = lax.dot_general(
        xs, w_ref[...], (((1,), (0,)), ((), ())),
        precision=lax.Precision.HIGHEST, preferred_element_type=jnp.float32)


def _matmul_halves(x, dinv2, W):
    return pl.pallas_call(
        _mm_body,
        grid=(N_NODES // _MM_R, NC),
        in_specs=[
            pl.BlockSpec((_MM_R, NFEAT), lambda i, j: (i, 0)),
            pl.BlockSpec((_MM_R, 1), lambda i, j: (i, 0)),
            pl.BlockSpec((NFEAT, HALF), lambda i, j: (0, j)),
        ],
        out_specs=pl.BlockSpec((1, _MM_R, HALF), lambda i, j: (j, i, 0)),
        out_shape=jax.ShapeDtypeStruct((NC, N_NODES, HALF), jnp.float32),
    )(x, dinv2, W)


def _scatter_body(h2_hbm, eidx_hbm, out_hbm, sb0, sb1, db0, db1,
                  rows0, rows1, acc_sh,
                  ssem0, ssem1, dsem0, dsem1, gsem0, gsem1):
    cid = lax.axis_index("c")
    sid = lax.axis_index("s")
    _copy_rows(sid, lambda o, n: h2_hbm.at[cid].at[pl.ds(o, n)],
               lambda o, n: acc_sh.at[pl.ds(o, n)])
    plsc.subcore_barrier()

    n = (N_CHUNKS - sid + NS - 1) // NS
    sbs, ssems = (sb0, sb1), (ssem0, ssem1)
    dbs, dsems = (db0, db1), (dsem0, dsem1)
    rows, gsems = (rows0, rows1), (gsem0, gsem1)

    def col(i):
        return (sid + i * NS) * CHUNK

    def issue_idx(i, s):
        @pl.when(i < n)
        def _():
            pltpu.async_copy(eidx_hbm.at[pl.ds(col(i), CHUNK)],
                             sbs[s], ssems[s])
            pltpu.async_copy(eidx_hbm.at[pl.ds(E + col(i), CHUNK)],
                             dbs[s], dsems[s])

    def step(i, s):
        @pl.when(i < n)
        def _():
            pltpu.make_async_copy(eidx_hbm.at[pl.ds(col(i), CHUNK)],
                                  sbs[s], ssems[s]).wait()
            pltpu.async_copy(h2_hbm.at[cid].at[sbs[s]],
                             rows[s], gsems[s])

        @pl.when((i >= 1) & (i - 1 < n))
        def _():
            pltpu.make_async_copy(h2_hbm.at[cid].at[sbs[1 - s]],
                                  rows[1 - s], gsems[1 - s]).wait()
            pltpu.make_async_copy(eidx_hbm.at[pl.ds(E + col(i - 1), CHUNK)],
                                  dbs[1 - s], dsems[1 - s]).wait()
            pltpu.sync_copy(rows[1 - s], acc_sh.at[dbs[1 - s]],
                            add=True)
            issue_idx(i + 1, 1 - s)

    issue_idx(0, 0)
    issue_idx(1, 1)

    @pl.loop(0, 40)
    def _(k):
        step(k * 2, 0)
        step(k * 2 + 1, 1)

    plsc.subcore_barrier()
    _copy_rows(sid, lambda o, n: acc_sh.at[pl.ds(o, n)],
               lambda o, n: out_hbm.at[cid].at[pl.ds(o, n)])


def _gather_scatter(h2, eidx):
    k = pl.kernel(
        _scatter_body,
        out_type=jax.ShapeDtypeStruct((NC, N_NODES, HALF), jnp.float32),
        mesh=_sc_mesh,
        scratch_types=[
            pltpu.VMEM((CHUNK,), jnp.int32),
            pltpu.VMEM((CHUNK,), jnp.int32),
            pltpu.VMEM((CHUNK,), jnp.int32),
            pltpu.VMEM((CHUNK,), jnp.int32),
            pltpu.VMEM((CHUNK, HALF), jnp.float32),
            pltpu.VMEM((CHUNK, HALF), jnp.float32),
            pltpu.VMEM_SHARED((N_NODES, HALF), jnp.float32),
            pltpu.SemaphoreType.DMA,
            pltpu.SemaphoreType.DMA,
            pltpu.SemaphoreType.DMA,
            pltpu.SemaphoreType.DMA,
            pltpu.SemaphoreType.DMA,
            pltpu.SemaphoreType.DMA,
        ],
    )
    return k(h2, eidx)


_EP_R = 2000


def _ep_body(acc_ref, dinv_ref, b_ref, o_ref):
    a = jnp.concatenate([acc_ref[0], acc_ref[1]], axis=1)
    o_ref[...] = jnp.maximum(a * dinv_ref[...] + b_ref[...], 0.0)


def _epilogue(acc, dinv2, b2):
    return pl.pallas_call(
        _ep_body,
        grid=(N_NODES // _EP_R,),
        in_specs=[
            pl.BlockSpec((NC, _EP_R, HALF), lambda i: (0, i, 0)),
            pl.BlockSpec((_EP_R, 1), lambda i: (i, 0)),
            pl.BlockSpec((1, NHID), lambda i: (0, 0)),
        ],
        out_specs=pl.BlockSpec((_EP_R, NHID), lambda i: (i, 0)),
        out_shape=jax.ShapeDtypeStruct((N_NODES, NHID), jnp.float32),
    )(acc, dinv2, b2)


def kernel(x, edge_index, W, b):
    eidx = edge_index.astype(jnp.int32).reshape(-1)

    deg_parts = _degree_parts(eidx)
    deg = deg_parts[:, :, 0].sum(axis=0) + 1.0
    dinv2 = (deg ** -0.5)[:, None]

    h2 = _matmul_halves(x, dinv2, W)
    acc = _gather_scatter(h2, eidx)
    return _epilogue(acc, dinv2, b.reshape(1, NHID))
